# Initial kernel scaffold; baseline (speedup 1.0000x reference)
import functools

import jax
import jax.numpy as jnp
from jax import lax
from jax.experimental import pallas as pl
from jax.experimental.pallas import tpu as pltpu
from jax.experimental.pallas import tpu_sc as plsc

F, B, L, V, D = 26, 1024, 20, 100000, 128
N = F * B * L
NC, NS = 2, 16
NW = NC * NS
LANES = 16

IDX_COLS = 128
IDX_ROWS = N // IDX_COLS

EC_K = 5
EC_OUTER = (N // NW) // (EC_K * IDX_COLS)
CHUNK = EC_K * IDX_COLS

NB = 32
UNITS_PER_F = B // NB
UNITS = F * UNITS_PER_F
UNITS_PER_W = UNITS // NW
ROWS_PER_F = (B * L) // IDX_COLS


def _sc_body(idx_hbm, ebc1_hbm, ec1_hbm, ebc2_hbm, ec2_hbm,
             o_ebc1, o_ec1, o_ebc2, o_ec2,
             idx_v, rows_v, acc_v, sem):
    wid = lax.axis_index("s") * NC + lax.axis_index("c")

    def gather_chunk(table_hbm, idx_row0):
        pltpu.sync_copy(idx_hbm.at[pl.ds(idx_row0, EC_K)], idx_v)
        copies = []
        for j in range(EC_K):
            c = pltpu.async_copy(table_hbm.at[idx_v.at[j]],
                                 rows_v.at[pl.ds(j * IDX_COLS, IDX_COLS)], sem)
            copies.append(c)
        for c in copies:
            c.wait()

    def ec_pass(table_hbm, out_hbm):
        def body(i, carry):
            idx_row0 = pl.multiple_of(wid * (EC_OUTER * EC_K) + i * EC_K, EC_K)
            gather_chunk(table_hbm, idx_row0)
            out_row0 = pl.multiple_of(idx_row0 * IDX_COLS, CHUNK)
            pltpu.sync_copy(rows_v, out_hbm.at[pl.ds(out_row0, CHUNK)])
            return carry
        lax.fori_loop(0, EC_OUTER, body, 0)

    def ebc_pass(table_hbm, out_hbm):
        def body(u, carry):
            unit = wid * UNITS_PER_W + u
            f = unit // UNITS_PER_F
            bc = unit % UNITS_PER_F
            idx_row0 = pl.multiple_of(f * ROWS_PER_F + bc * EC_K, EC_K)
            gather_chunk(table_hbm, idx_row0)

            def pool(bag, c2):
                base = bag * L
                for c in range(D // LANES):
                    acc = rows_v[base, pl.ds(c * LANES, LANES)]
                    for l in range(1, L):
                        acc = acc + rows_v[base + l, pl.ds(c * LANES, LANES)]
                    acc_v[bag, pl.ds(c * LANES, LANES)] = acc
                return c2
            lax.fori_loop(0, NB, pool, 0)

            b0 = pl.multiple_of(bc * NB, NB)
            col0 = pl.multiple_of(f * D, D)
            pltpu.sync_copy(acc_v, out_hbm.at[pl.ds(b0, NB), pl.ds(col0, D)])
            return carry
        lax.fori_loop(0, UNITS_PER_W, body, 0)

    ec_pass(ec1_hbm, o_ec1)
    ec_pass(ec2_hbm, o_ec2)
    ebc_pass(ebc1_hbm, o_ebc1)
    ebc_pass(ebc2_hbm, o_ebc2)


@jax.jit
def kernel(indices, ebc1_table, ec1_table, ebc2_table, ec2_table):
    idx2d = indices.reshape(IDX_ROWS, IDX_COLS)
    mesh = plsc.VectorSubcoreMesh(core_axis_name="c", subcore_axis_name="s",
                                  num_cores=NC, num_subcores=NS)
    out_type = (
        jax.ShapeDtypeStruct((B, F * D), jnp.float32),
        jax.ShapeDtypeStruct((N, D), jnp.float32),
        jax.ShapeDtypeStruct((B, F * D), jnp.float32),
        jax.ShapeDtypeStruct((N, D), jnp.float32),
    )
    scratch = [
        pltpu.VMEM((EC_K, IDX_COLS), jnp.int32),
        pltpu.VMEM((CHUNK, D), jnp.float32),
        pltpu.VMEM((NB, D), jnp.float32),
        pltpu.SemaphoreType.DMA,
    ]
    fn = pl.kernel(_sc_body, out_type=out_type, mesh=mesh,
                   scratch_types=scratch)
    return fn(idx2d, ebc1_table, ec1_table, ebc2_table, ec2_table)


# speedup vs baseline: 9.2320x; 9.2320x over previous
import functools

import jax
import jax.numpy as jnp
from jax import lax
from jax.experimental import pallas as pl
from jax.experimental.pallas import tpu as pltpu
from jax.experimental.pallas import tpu_sc as plsc

F, B, L, V, D = 26, 1024, 20, 100000, 128
N = F * B * L
NC, NS = 2, 16
NW = NC * NS
LANES = 16

IDX_COLS = 128
IDX_ROWS = N // IDX_COLS

EC_K = 5
EC_OUTER = (N // NW) // (EC_K * IDX_COLS)
CHUNK = EC_K * IDX_COLS

NB = 32
UNITS_PER_F = B // NB
UNITS = F * UNITS_PER_F
UNITS_PER_W = UNITS // NW
ROWS_PER_F = (B * L) // IDX_COLS


def _sc_body(idx_hbm, ebc1_hbm, ec1_hbm, ebc2_hbm, ec2_hbm,
             o_ebc1, o_ec1, o_ebc2, o_ec2,
             idx_v, rows_v, acc_v, sem):
    wid = lax.axis_index("s") * NC + lax.axis_index("c")

    def gather_chunk(table_hbm, idx0):
        pltpu.sync_copy(idx_hbm.at[pl.ds(idx0, CHUNK)], idx_v)
        copies = []
        for j in range(EC_K):
            c = pltpu.async_copy(
                table_hbm.at[idx_v.at[pl.ds(j * IDX_COLS, IDX_COLS)]],
                rows_v.at[pl.ds(j * IDX_COLS, IDX_COLS)], sem)
            copies.append(c)
        for c in copies:
            c.wait()

    def ec_pass(table_hbm, out_hbm):
        def body(i, carry):
            idx0 = pl.multiple_of(wid * (EC_OUTER * CHUNK) + i * CHUNK, CHUNK)
            gather_chunk(table_hbm, idx0)
            pltpu.sync_copy(rows_v, out_hbm.at[pl.ds(idx0, CHUNK)])
            return carry
        lax.fori_loop(0, EC_OUTER, body, 0)

    def ebc_pass(table_hbm, out_hbm):
        def body(u, carry):
            unit = wid * UNITS_PER_W + u
            f = unit // UNITS_PER_F
            bc = unit % UNITS_PER_F
            idx0 = pl.multiple_of(f * (B * L) + bc * CHUNK, CHUNK)
            gather_chunk(table_hbm, idx0)

            def pool(bag, c2):
                base = bag * L
                for c in range(D // LANES):
                    acc = rows_v[base, pl.ds(c * LANES, LANES)]
                    for l in range(1, L):
                        acc = acc + rows_v[base + l, pl.ds(c * LANES, LANES)]
                    acc_v[bag, pl.ds(c * LANES, LANES)] = acc
                return c2
            lax.fori_loop(0, NB, pool, 0)

            b0 = pl.multiple_of(bc * NB, NB)
            col0 = pl.multiple_of(f * D, D)
            pltpu.sync_copy(acc_v, out_hbm.at[pl.ds(b0, NB), pl.ds(col0, D)])
            return carry
        lax.fori_loop(0, UNITS_PER_W, body, 0)

    ec_pass(ec1_hbm, o_ec1)
    ec_pass(ec2_hbm, o_ec2)
    ebc_pass(ebc1_hbm, o_ebc1)
    ebc_pass(ebc2_hbm, o_ebc2)


@jax.jit
def kernel(indices, ebc1_table, ec1_table, ebc2_table, ec2_table):
    idx1d = indices.reshape(N)
    mesh = plsc.VectorSubcoreMesh(core_axis_name="c", subcore_axis_name="s",
                                  num_cores=NC, num_subcores=NS)
    out_type = (
        jax.ShapeDtypeStruct((B, F * D), jnp.float32),
        jax.ShapeDtypeStruct((N, D), jnp.float32),
        jax.ShapeDtypeStruct((B, F * D), jnp.float32),
        jax.ShapeDtypeStruct((N, D), jnp.float32),
    )
    scratch = [
        pltpu.VMEM((CHUNK,), jnp.int32),
        pltpu.VMEM((CHUNK, D), jnp.float32),
        pltpu.VMEM((NB, D), jnp.float32),
        pltpu.SemaphoreType.DMA,
    ]
    fn = pl.kernel(_sc_body, out_type=out_type, mesh=mesh,
                   scratch_types=scratch)
    return fn(idx1d, ebc1_table, ec1_table, ebc2_table, ec2_table)


# speedup vs baseline: 11.4301x; 1.2381x over previous
import jax
import jax.numpy as jnp
from jax import lax
from jax.experimental import pallas as pl
from jax.experimental.pallas import tpu as pltpu
from jax.experimental.pallas import tpu_sc as plsc

F, B, L, V, D = 26, 1024, 20, 100000, 128
N = F * B * L
NC, NS = 2, 16
NW = NC * NS
LANES = 16

CHUNK = 320
SLICES = ((0, 128), (128, 128), (256, 64))
NCHUNK = (N // NW) // CHUNK
BAGS = CHUNK // L
UNITS_PER_F = B // BAGS
ROWS_PER_W = N // NW


def _sc_body(idx_hbm, ebc1_hbm, ec1_hbm, ebc2_hbm, ec2_hbm,
             o_ebc1, o_ec1, o_ebc2, o_ec2,
             rows0, rows1, idx0, idx1, acc0, acc1,
             gsem0, gsem1, wsem0, wsem1):
    wid = lax.axis_index("s") * NC + lax.axis_index("c")
    rows = (rows0, rows1)
    idxb = (idx0, idx1)
    accb = (acc0, acc1)
    gsem = (gsem0, gsem1)
    wsem = (wsem0, wsem1)

    def fire(p, i0, table_hbm):
        pltpu.sync_copy(idx_hbm.at[pl.ds(i0, CHUNK)], idxb[p])
        for (off, sz) in SLICES:
            pltpu.async_copy(table_hbm.at[idxb[p].at[pl.ds(off, sz)]],
                             rows[p].at[pl.ds(off, sz)], gsem[p])

    def drain_gather(p, table_hbm):
        pltpu.make_async_copy(table_hbm.at[pl.ds(0, CHUNK)], rows[p],
                              gsem[p]).wait()

    def run_pass(table_hbm, idx0_of, process, drain_write):
        fire(0, idx0_of(0), table_hbm)

        def outer(i2, carry):
            for p in (0, 1):
                i = i2 * 2 + p

                @pl.when(i + 1 < NCHUNK)
                def _():
                    @pl.when(i >= 1)
                    def _():
                        drain_write(1 - p)
                    fire(1 - p, idx0_of(i + 1), table_hbm)

                drain_gather(p, table_hbm)
                process(p, i)
            return carry

        lax.fori_loop(0, NCHUNK // 2, outer, 0)
        drain_write(0)
        drain_write(1)

    def ec_pass(table_hbm, out_hbm):
        def idx0_of(i):
            return pl.multiple_of(wid * ROWS_PER_W + i * CHUNK, CHUNK)

        def process(p, i):
            pltpu.async_copy(rows[p], out_hbm.at[pl.ds(idx0_of(i), CHUNK)],
                             wsem[p])

        def drain_write(q):
            pltpu.make_async_copy(rows[q], out_hbm.at[pl.ds(0, CHUNK)],
                                  wsem[q]).wait()

        run_pass(table_hbm, idx0_of, process, drain_write)

    def ebc_pass(table_hbm, out_hbm):
        def unit_of(i):
            return wid * NCHUNK + i

        def idx0_of(i):
            u = unit_of(i)
            f = u // UNITS_PER_F
            bc = u % UNITS_PER_F
            return pl.multiple_of(f * (B * L) + bc * CHUNK, CHUNK)

        def process(p, i):
            def pool(bag, c2):
                base = bag * L
                for c in range(D // LANES):
                    acc = rows[p][base, pl.ds(c * LANES, LANES)]
                    for l in range(1, L):
                        acc = acc + rows[p][base + l, pl.ds(c * LANES, LANES)]
                    accb[p][bag, pl.ds(c * LANES, LANES)] = acc
                return c2

            lax.fori_loop(0, BAGS, pool, 0)
            u = unit_of(i)
            f = u // UNITS_PER_F
            bc = u % UNITS_PER_F
            b0 = pl.multiple_of(bc * BAGS, BAGS)
            col0 = pl.multiple_of(f * D, D)
            pltpu.async_copy(accb[p],
                             out_hbm.at[pl.ds(b0, BAGS), pl.ds(col0, D)],
                             wsem[p])

        def drain_write(q):
            pltpu.make_async_copy(accb[q],
                                  out_hbm.at[pl.ds(0, BAGS), pl.ds(0, D)],
                                  wsem[q]).wait()

        run_pass(table_hbm, idx0_of, process, drain_write)

    ec_pass(ec1_hbm, o_ec1)
    ec_pass(ec2_hbm, o_ec2)
    ebc_pass(ebc1_hbm, o_ebc1)
    ebc_pass(ebc2_hbm, o_ebc2)


@jax.jit
def kernel(indices, ebc1_table, ec1_table, ebc2_table, ec2_table):
    idx1d = indices.reshape(N)
    mesh = plsc.VectorSubcoreMesh(core_axis_name="c", subcore_axis_name="s",
                                  num_cores=NC, num_subcores=NS)
    out_type = (
        jax.ShapeDtypeStruct((B, F * D), jnp.float32),
        jax.ShapeDtypeStruct((N, D), jnp.float32),
        jax.ShapeDtypeStruct((B, F * D), jnp.float32),
        jax.ShapeDtypeStruct((N, D), jnp.float32),
    )
    scratch = [
        pltpu.VMEM((CHUNK, D), jnp.float32),
        pltpu.VMEM((CHUNK, D), jnp.float32),
        pltpu.VMEM((CHUNK,), jnp.int32),
        pltpu.VMEM((CHUNK,), jnp.int32),
        pltpu.VMEM((BAGS, D), jnp.float32),
        pltpu.VMEM((BAGS, D), jnp.float32),
        pltpu.SemaphoreType.DMA,
        pltpu.SemaphoreType.DMA,
        pltpu.SemaphoreType.DMA,
        pltpu.SemaphoreType.DMA,
    ]
    fn = pl.kernel(_sc_body, out_type=out_type, mesh=mesh,
                   scratch_types=scratch)
    return fn(idx1d, ebc1_table, ec1_table, ebc2_table, ec2_table)


# speedup vs baseline: 12.3316x; 1.0789x over previous
import jax
import jax.numpy as jnp
from jax import lax
from jax.experimental import pallas as pl
from jax.experimental.pallas import tpu as pltpu
from jax.experimental.pallas import tpu_sc as plsc

F, B, L, V, D = 26, 1024, 20, 100000, 128
N = F * B * L
NC, NS = 2, 16
NW = NC * NS
LANES = 16

CHUNK = 320
SLICES = ((0, 128), (128, 128), (256, 64))
NCHUNK = (N // NW) // CHUNK
BAGS = CHUNK // L
UNITS_PER_F = B // BAGS
ROWS_PER_W = N // NW


def _sc_body(idx_hbm, ebc1_hbm, ec1_hbm, ebc2_hbm, ec2_hbm,
             o_ebc1, o_ec1, o_ebc2, o_ec2,
             rows0, rows1, idx_all, acc0, acc1,
             gsem0, gsem1, wsem0, wsem1):
    wid = lax.axis_index("s") * NC + lax.axis_index("c")
    rows = (rows0, rows1)
    accb = (acc0, acc1)
    gsem = (gsem0, gsem1)
    wsem = (wsem0, wsem1)

    pltpu.sync_copy(idx_hbm.at[pl.ds(wid * ROWS_PER_W, ROWS_PER_W)], idx_all)

    def fire(p, i, table_hbm):
        for (off, sz) in SLICES:
            o = pl.multiple_of(i * CHUNK + off, 8)
            pltpu.async_copy(table_hbm.at[idx_all.at[pl.ds(o, sz)]],
                             rows[p].at[pl.ds(off, sz)], gsem[p])

    def drain_gather(p, table_hbm):
        pltpu.make_async_copy(table_hbm.at[pl.ds(0, CHUNK)], rows[p],
                              gsem[p]).wait()

    def run_pass(table_hbm, process, drain_write):
        fire(0, 0, table_hbm)

        def outer(i2, carry):
            for p in (0, 1):
                i = i2 * 2 + p

                @pl.when(i + 1 < NCHUNK)
                def _():
                    @pl.when(i >= 1)
                    def _():
                        drain_write(1 - p)
                    fire(1 - p, i + 1, table_hbm)

                drain_gather(p, table_hbm)
                process(p, i)
            return carry

        lax.fori_loop(0, NCHUNK // 2, outer, 0)
        drain_write(0)
        drain_write(1)

    def ec_pass(table_hbm, out_hbm):
        def process(p, i):
            o0 = pl.multiple_of(wid * ROWS_PER_W + i * CHUNK, CHUNK)
            pltpu.async_copy(rows[p], out_hbm.at[pl.ds(o0, CHUNK)], wsem[p])

        def drain_write(q):
            pltpu.make_async_copy(rows[q], out_hbm.at[pl.ds(0, CHUNK)],
                                  wsem[q]).wait()

        run_pass(table_hbm, process, drain_write)

    def ebc_pass(table_hbm, out_hbm):
        def unit_of(i):
            return wid * NCHUNK + i

        def process(p, i):
            def pool(bag, c2):
                base = bag * L
                for c in range(D // LANES):
                    acc = rows[p][base, pl.ds(c * LANES, LANES)]
                    for l in range(1, L):
                        acc = acc + rows[p][base + l, pl.ds(c * LANES, LANES)]
                    accb[p][bag, pl.ds(c * LANES, LANES)] = acc
                return c2

            lax.fori_loop(0, BAGS, pool, 0)
            u = unit_of(i)
            f = u // UNITS_PER_F
            bc = u % UNITS_PER_F
            b0 = pl.multiple_of(bc * BAGS, BAGS)
            col0 = pl.multiple_of(f * D, D)
            pltpu.async_copy(accb[p],
                             out_hbm.at[pl.ds(b0, BAGS), pl.ds(col0, D)],
                             wsem[p])

        def drain_write(q):
            pltpu.make_async_copy(accb[q],
                                  out_hbm.at[pl.ds(0, BAGS), pl.ds(0, D)],
                                  wsem[q]).wait()

        run_pass(table_hbm, process, drain_write)

    ec_pass(ec1_hbm, o_ec1)
    ec_pass(ec2_hbm, o_ec2)
    ebc_pass(ebc1_hbm, o_ebc1)
    ebc_pass(ebc2_hbm, o_ebc2)


@jax.jit
def kernel(indices, ebc1_table, ec1_table, ebc2_table, ec2_table):
    idx1d = indices.reshape(N)
    mesh = plsc.VectorSubcoreMesh(core_axis_name="c", subcore_axis_name="s",
                                  num_cores=NC, num_subcores=NS)
    out_type = (
        jax.ShapeDtypeStruct((B, F * D), jnp.float32),
        jax.ShapeDtypeStruct((N, D), jnp.float32),
        jax.ShapeDtypeStruct((B, F * D), jnp.float32),
        jax.ShapeDtypeStruct((N, D), jnp.float32),
    )
    scratch = [
        pltpu.VMEM((CHUNK, D), jnp.float32),
        pltpu.VMEM((CHUNK, D), jnp.float32),
        pltpu.VMEM((ROWS_PER_W,), jnp.int32),
        pltpu.VMEM((BAGS, D), jnp.float32),
        pltpu.VMEM((BAGS, D), jnp.float32),
        pltpu.SemaphoreType.DMA,
        pltpu.SemaphoreType.DMA,
        pltpu.SemaphoreType.DMA,
        pltpu.SemaphoreType.DMA,
    ]
    fn = pl.kernel(_sc_body, out_type=out_type, mesh=mesh,
                   scratch_types=scratch)
    return fn(idx1d, ebc1_table, ec1_table, ebc2_table, ec2_table)


# speedup vs baseline: 14.7033x; 1.1923x over previous
import jax
import jax.numpy as jnp
from jax import lax
from jax.experimental import pallas as pl
from jax.experimental.pallas import tpu as pltpu
from jax.experimental.pallas import tpu_sc as plsc

F, B, L, V, D = 26, 1024, 20, 100000, 128
N = F * B * L
NC, NS = 2, 16
NW = NC * NS
LANES = 16

CHUNK = 320
SLICES = ((0, 128), (128, 128), (256, 64))
NCHUNK = (N // NW) // CHUNK
BAGS = CHUNK // L
UNITS_PER_F = B // BAGS
ROWS_PER_W = N // NW


def _sc_body(idx_hbm, ebc1_hbm, ec1_hbm, ebc2_hbm, ec2_hbm,
             o_ebc1, o_ec1, o_ebc2, o_ec2,
             rows0, rows1, idx_all, acc0, acc1,
             gsem0, gsem1, wsem0, wsem1):
    wid = lax.axis_index("s") * NC + lax.axis_index("c")
    rows = (rows0, rows1)
    accb = (acc0, acc1)
    gsem = (gsem0, gsem1)
    wsem = (wsem0, wsem1)

    pltpu.sync_copy(idx_hbm.at[pl.ds(wid * ROWS_PER_W, ROWS_PER_W)], idx_all)

    def fire(p, i, table_hbm):
        for (off, sz) in SLICES:
            o = pl.multiple_of(i * CHUNK + off, 8)
            pltpu.async_copy(table_hbm.at[idx_all.at[pl.ds(o, sz)]],
                             rows[p].at[pl.ds(off, sz)], gsem[p])

    def drain_gather(p, table_hbm):
        pltpu.make_async_copy(table_hbm.at[pl.ds(0, CHUNK)], rows[p],
                              gsem[p]).wait()

    def process_ec(p, i, out_hbm):
        o0 = pl.multiple_of(wid * ROWS_PER_W + i * CHUNK, CHUNK)
        pltpu.async_copy(rows[p], out_hbm.at[pl.ds(o0, CHUNK)], wsem[p])

    def drain_write_ec(q, out_hbm):
        pltpu.make_async_copy(rows[q], out_hbm.at[pl.ds(0, CHUNK)],
                              wsem[q]).wait()

    def process_ebc(p, i, out_hbm):
        def pool(bag, c2):
            base = bag * L
            for c in range(D // LANES):
                acc = rows[p][base, pl.ds(c * LANES, LANES)]
                for l in range(1, L):
                    acc = acc + rows[p][base + l, pl.ds(c * LANES, LANES)]
                accb[p][bag, pl.ds(c * LANES, LANES)] = acc
            return c2

        lax.fori_loop(0, BAGS, pool, 0)
        u = wid * NCHUNK + i
        f = u // UNITS_PER_F
        bc = u % UNITS_PER_F
        b0 = pl.multiple_of(bc * BAGS, BAGS)
        col0 = pl.multiple_of(f * D, D)
        pltpu.async_copy(accb[p],
                         out_hbm.at[pl.ds(b0, BAGS), pl.ds(col0, D)],
                         wsem[p])

    def drain_write_ebc(q, out_hbm):
        pltpu.make_async_copy(accb[q],
                              out_hbm.at[pl.ds(0, BAGS), pl.ds(0, D)],
                              wsem[q]).wait()

    STEPS = (
        (ec1_hbm, o_ec1, process_ec, drain_write_ec),
        (ec2_hbm, o_ec2, process_ec, drain_write_ec),
        (ebc1_hbm, o_ebc1, process_ebc, drain_write_ebc),
        (ebc2_hbm, o_ebc2, process_ebc, drain_write_ebc),
    )

    fire(0, 0, STEPS[0][0])

    def outer(i, carry):
        for t in range(4):
            p = t % 2
            nt = (t + 1) % 4
            pt = (t - 1) % 4
            prev_out, prev_drain = STEPS[pt][1], STEPS[pt][3]
            ni = i + 1 if t == 3 else i

            if t == 0:
                @pl.when(i >= 1)
                def _():
                    prev_drain(1 - p, prev_out)
                fire(1 - p, ni, STEPS[nt][0])
            elif t == 3:
                prev_drain(1 - p, prev_out)

                @pl.when(i + 1 < NCHUNK)
                def _():
                    fire(1 - p, ni, STEPS[nt][0])
            else:
                prev_drain(1 - p, prev_out)
                fire(1 - p, ni, STEPS[nt][0])

            drain_gather(p, STEPS[t][0])
            STEPS[t][2](p, i, STEPS[t][1])
        return carry

    lax.fori_loop(0, NCHUNK, outer, 0)
    drain_write_ebc(1, o_ebc2)


@jax.jit
def kernel(indices, ebc1_table, ec1_table, ebc2_table, ec2_table):
    idx1d = indices.reshape(N)
    mesh = plsc.VectorSubcoreMesh(core_axis_name="c", subcore_axis_name="s",
                                  num_cores=NC, num_subcores=NS)
    out_type = (
        jax.ShapeDtypeStruct((B, F * D), jnp.float32),
        jax.ShapeDtypeStruct((N, D), jnp.float32),
        jax.ShapeDtypeStruct((B, F * D), jnp.float32),
        jax.ShapeDtypeStruct((N, D), jnp.float32),
    )
    scratch = [
        pltpu.VMEM((CHUNK, D), jnp.float32),
        pltpu.VMEM((CHUNK, D), jnp.float32),
        pltpu.VMEM((ROWS_PER_W,), jnp.int32),
        pltpu.VMEM((BAGS, D), jnp.float32),
        pltpu.VMEM((BAGS, D), jnp.float32),
        pltpu.SemaphoreType.DMA,
        pltpu.SemaphoreType.DMA,
        pltpu.SemaphoreType.DMA,
        pltpu.SemaphoreType.DMA,
    ]
    fn = pl.kernel(_sc_body, out_type=out_type, mesh=mesh,
                   scratch_types=scratch)
    return fn(idx1d, ebc1_table, ec1_table, ebc2_table, ec2_table)


# speedup vs baseline: 14.7132x; 1.0007x over previous
import jax
import jax.numpy as jnp
from jax import lax
from jax.experimental import pallas as pl
from jax.experimental.pallas import tpu as pltpu
from jax.experimental.pallas import tpu_sc as plsc

F, B, L, V, D = 26, 1024, 20, 100000, 128
N = F * B * L
NC, NS = 2, 16
NW = NC * NS
LANES = 16

CHUNK = 160
SLICES = ((0, 80), (80, 80))
NCHUNK = (N // NW) // CHUNK
BAGS = CHUNK // L
UNITS_PER_F = B // BAGS
ROWS_PER_W = N // NW
NT = 4
AHEAD = 2


def _sc_body(idx_hbm, ebc1_hbm, ec1_hbm, ebc2_hbm, ec2_hbm,
             o_ebc1, o_ec1, o_ebc2, o_ec2,
             rows0, rows1, rows2, rows3, idx_all, acc0, acc1,
             gsem0, gsem1, gsem2, gsem3, wsem0, wsem1, wsem2, wsem3):
    wid = lax.axis_index("s") * NC + lax.axis_index("c")
    rows = (rows0, rows1, rows2, rows3)
    accb = {2: acc0, 3: acc1}
    gsem = (gsem0, gsem1, gsem2, gsem3)
    wsem = (wsem0, wsem1, wsem2, wsem3)

    pltpu.sync_copy(idx_hbm.at[pl.ds(wid * ROWS_PER_W, ROWS_PER_W)], idx_all)

    tables = (ec1_hbm, ec2_hbm, ebc1_hbm, ebc2_hbm)
    outs = (o_ec1, o_ec2, o_ebc1, o_ebc2)

    def fire(b, i, table_hbm):
        for (off, sz) in SLICES:
            o = pl.multiple_of(i * CHUNK + off, 8)
            pltpu.async_copy(table_hbm.at[idx_all.at[pl.ds(o, sz)]],
                             rows[b].at[pl.ds(off, sz)], gsem[b])

    def drain_gather(b, table_hbm):
        pltpu.make_async_copy(table_hbm.at[pl.ds(0, CHUNK)], rows[b],
                              gsem[b]).wait()

    def process(t, i):
        out_hbm = outs[t]
        if t < 2:
            o0 = pl.multiple_of(wid * ROWS_PER_W + i * CHUNK, CHUNK)
            pltpu.async_copy(rows[t], out_hbm.at[pl.ds(o0, CHUNK)], wsem[t])
        else:
            acc = accb[t]

            def pool(bag, c2):
                base = bag * L
                for c in range(D // LANES):
                    a = rows[t][base, pl.ds(c * LANES, LANES)]
                    for l in range(1, L):
                        a = a + rows[t][base + l, pl.ds(c * LANES, LANES)]
                    acc[bag, pl.ds(c * LANES, LANES)] = a
                return c2

            lax.fori_loop(0, BAGS, pool, 0)
            u = wid * NCHUNK + i
            f = u // UNITS_PER_F
            bc = u % UNITS_PER_F
            b0 = pl.multiple_of(bc * BAGS, BAGS)
            col0 = pl.multiple_of(f * D, D)
            pltpu.async_copy(acc, out_hbm.at[pl.ds(b0, BAGS), pl.ds(col0, D)],
                             wsem[t])

    def drain_write(t):
        out_hbm = outs[t]
        if t < 2:
            pltpu.make_async_copy(rows[t], out_hbm.at[pl.ds(0, CHUNK)],
                                  wsem[t]).wait()
        else:
            pltpu.make_async_copy(accb[t],
                                  out_hbm.at[pl.ds(0, BAGS), pl.ds(0, D)],
                                  wsem[t]).wait()

    for s in range(AHEAD):
        fire(s % NT, 0, tables[s % NT])

    def outer(i, carry):
        for t in range(NT):
            ft = (t + AHEAD) % NT
            fi = i + 1 if t + AHEAD >= NT else i

            if t + AHEAD < NT:
                @pl.when(i >= 1)
                def _():
                    drain_write(ft)
                fire(ft, fi, tables[ft])
            else:
                drain_write(ft)

                @pl.when(fi < NCHUNK)
                def _():
                    fire(ft, fi, tables[ft])

            drain_gather(t, tables[t])
            process(t, i)
        return carry

    lax.fori_loop(0, NCHUNK, outer, 0)
    for t in range(NT - AHEAD, NT):
        drain_write(t)


@jax.jit
def kernel(indices, ebc1_table, ec1_table, ebc2_table, ec2_table):
    idx1d = indices.reshape(N)
    mesh = plsc.VectorSubcoreMesh(core_axis_name="c", subcore_axis_name="s",
                                  num_cores=NC, num_subcores=NS)
    out_type = (
        jax.ShapeDtypeStruct((B, F * D), jnp.float32),
        jax.ShapeDtypeStruct((N, D), jnp.float32),
        jax.ShapeDtypeStruct((B, F * D), jnp.float32),
        jax.ShapeDtypeStruct((N, D), jnp.float32),
    )
    scratch = [
        pltpu.VMEM((CHUNK, D), jnp.float32),
        pltpu.VMEM((CHUNK, D), jnp.float32),
        pltpu.VMEM((CHUNK, D), jnp.float32),
        pltpu.VMEM((CHUNK, D), jnp.float32),
        pltpu.VMEM((ROWS_PER_W,), jnp.int32),
        pltpu.VMEM((BAGS, D), jnp.float32),
        pltpu.VMEM((BAGS, D), jnp.float32),
        pltpu.SemaphoreType.DMA,
        pltpu.SemaphoreType.DMA,
        pltpu.SemaphoreType.DMA,
        pltpu.SemaphoreType.DMA,
        pltpu.SemaphoreType.DMA,
        pltpu.SemaphoreType.DMA,
        pltpu.SemaphoreType.DMA,
        pltpu.SemaphoreType.DMA,
    ]
    fn = pl.kernel(_sc_body, out_type=out_type, mesh=mesh,
                   scratch_types=scratch)
    return fn(idx1d, ebc1_table, ec1_table, ebc2_table, ec2_table)


# speedup vs baseline: 15.9126x; 1.0815x over previous
import jax
import jax.numpy as jnp
from jax import lax
from jax.experimental import pallas as pl
from jax.experimental.pallas import tpu as pltpu
from jax.experimental.pallas import tpu_sc as plsc

F, B, L, V, D = 26, 1024, 20, 100000, 128
N = F * B * L
NC, NS = 2, 16
NW = NC * NS
LANES = 16

CHUNK = 160
SLICES = ((0, 80), (80, 80))
NCHUNK = (N // NW) // CHUNK
BAGS = CHUNK // L
UNITS_PER_F = B // BAGS
ROWS_PER_W = N // NW
NT = 4
AHEAD = 2


def _sc_body(idx_hbm, ebc1_hbm, ec1_hbm, ebc2_hbm, ec2_hbm,
             o_ebc1, o_ec1, o_ebc2, o_ec2,
             rows0, rows1, rows2, rows3, idx_all, acc0, acc1,
             gsem0, gsem1, gsem2, gsem3, wsem0, wsem1, wsem2, wsem3):
    wid = lax.axis_index("s") * NC + lax.axis_index("c")
    rows = (rows0, rows1, rows2, rows3)
    accb = {1: acc0, 3: acc1}
    gsem = (gsem0, gsem1, gsem2, gsem3)
    wsem = (wsem0, wsem1, wsem2, wsem3)

    pltpu.sync_copy(idx_hbm.at[pl.ds(wid * ROWS_PER_W, ROWS_PER_W)], idx_all)

    tables = (ec1_hbm, ebc1_hbm, ec2_hbm, ebc2_hbm)
    outs = (o_ec1, o_ebc1, o_ec2, o_ebc2)

    def fire(b, i, table_hbm):
        for (off, sz) in SLICES:
            o = pl.multiple_of(i * CHUNK + off, 8)
            pltpu.async_copy(table_hbm.at[idx_all.at[pl.ds(o, sz)]],
                             rows[b].at[pl.ds(off, sz)], gsem[b])

    def drain_gather(b, table_hbm):
        pltpu.make_async_copy(table_hbm.at[pl.ds(0, CHUNK)], rows[b],
                              gsem[b]).wait()

    def process(t, i):
        out_hbm = outs[t]
        if t % 2 == 0:
            o0 = pl.multiple_of(wid * ROWS_PER_W + i * CHUNK, CHUNK)
            pltpu.async_copy(rows[t], out_hbm.at[pl.ds(o0, CHUNK)], wsem[t])
        else:
            acc = accb[t]

            def pool(bag, c2):
                base = bag * L
                for c in range(D // LANES):
                    a = rows[t][base, pl.ds(c * LANES, LANES)]
                    for l in range(1, L):
                        a = a + rows[t][base + l, pl.ds(c * LANES, LANES)]
                    acc[bag, pl.ds(c * LANES, LANES)] = a
                return c2

            lax.fori_loop(0, BAGS, pool, 0)
            u = wid * NCHUNK + i
            f = u // UNITS_PER_F
            bc = u % UNITS_PER_F
            b0 = pl.multiple_of(bc * BAGS, BAGS)
            col0 = pl.multiple_of(f * D, D)
            pltpu.async_copy(acc, out_hbm.at[pl.ds(b0, BAGS), pl.ds(col0, D)],
                             wsem[t])

    def drain_write(t):
        out_hbm = outs[t]
        if t % 2 == 0:
            pltpu.make_async_copy(rows[t], out_hbm.at[pl.ds(0, CHUNK)],
                                  wsem[t]).wait()
        else:
            pltpu.make_async_copy(accb[t],
                                  out_hbm.at[pl.ds(0, BAGS), pl.ds(0, D)],
                                  wsem[t]).wait()

    for s in range(AHEAD):
        fire(s % NT, 0, tables[s % NT])

    def outer(i, carry):
        for t in range(NT):
            ft = (t + AHEAD) % NT
            fi = i + 1 if t + AHEAD >= NT else i

            if t + AHEAD < NT:
                @pl.when(i >= 1)
                def _():
                    drain_write(ft)
                fire(ft, fi, tables[ft])
            else:
                drain_write(ft)

                @pl.when(fi < NCHUNK)
                def _():
                    fire(ft, fi, tables[ft])

            drain_gather(t, tables[t])
            process(t, i)
        return carry

    lax.fori_loop(0, NCHUNK, outer, 0)
    for t in range(NT - AHEAD, NT):
        drain_write(t)


@jax.jit
def kernel(indices, ebc1_table, ec1_table, ebc2_table, ec2_table):
    idx1d = indices.reshape(N)
    mesh = plsc.VectorSubcoreMesh(core_axis_name="c", subcore_axis_name="s",
                                  num_cores=NC, num_subcores=NS)
    out_type = (
        jax.ShapeDtypeStruct((B, F * D), jnp.float32),
        jax.ShapeDtypeStruct((N, D), jnp.float32),
        jax.ShapeDtypeStruct((B, F * D), jnp.float32),
        jax.ShapeDtypeStruct((N, D), jnp.float32),
    )
    scratch = [
        pltpu.VMEM((CHUNK, D), jnp.float32),
        pltpu.VMEM((CHUNK, D), jnp.float32),
        pltpu.VMEM((CHUNK, D), jnp.float32),
        pltpu.VMEM((CHUNK, D), jnp.float32),
        pltpu.VMEM((ROWS_PER_W,), jnp.int32),
        pltpu.VMEM((BAGS, D), jnp.float32),
        pltpu.VMEM((BAGS, D), jnp.float32),
        pltpu.SemaphoreType.DMA,
        pltpu.SemaphoreType.DMA,
        pltpu.SemaphoreType.DMA,
        pltpu.SemaphoreType.DMA,
        pltpu.SemaphoreType.DMA,
        pltpu.SemaphoreType.DMA,
        pltpu.SemaphoreType.DMA,
        pltpu.SemaphoreType.DMA,
    ]
    fn = pl.kernel(_sc_body, out_type=out_type, mesh=mesh,
                   scratch_types=scratch)
    return fn(idx1d, ebc1_table, ec1_table, ebc2_table, ec2_table)
